# baseline (speedup 1.0000x reference)
import functools

import jax
import jax.numpy as jnp
from jax import lax
from jax.experimental import pallas as pl
from jax.experimental.pallas import tpu as pltpu
from jax.experimental.pallas import tpu_sc as plsc

_N_EDGES = 320000
_N_NODES = 10000
_D = 128
_W = _D // 2
_NW = 32
_EW = _N_EDGES // _NW
_C = 128
_NF = _EW // _C
_CR = _EW - _NF * _C
_G = _C // 16


def _body(eidx_hbm, emb1_hbm, emb2_hbm, out_hbm,
          idx1_v, idx2_v, a0, b0, a1, b1, out_v, sh1,
          sa0, sb0, sa1, sb1, st1, st2):
    nc = 2
    wid = lax.axis_index("s") * nc + lax.axis_index("c")
    base = wid * _EW

    sid = lax.axis_index("s")
    srows = _N_NODES // 16
    soff = sid * srows
    cs1 = pltpu.async_copy(emb1_hbm.at[pl.ds(soff, srows)],
                           sh1.at[pl.ds(soff, srows)], st1)

    pltpu.sync_copy(eidx_hbm.at[0, pl.ds(base, _EW)], idx1_v)
    pltpu.sync_copy(eidx_hbm.at[1, pl.ds(base, _EW)], idx2_v)

    cs1.wait()
    plsc.subcore_barrier()

    lane = lax.iota(jnp.int32, 16)

    def start(off, n, abuf, bbuf, sa, sb):
        pltpu.async_copy(sh1.at[idx1_v.at[pl.ds(off, n)]],
                         abuf.at[pl.ds(0, n)], sa)
        pltpu.async_copy(emb2_hbm.at[idx2_v.at[pl.ds(off, n)]],
                         bbuf.at[pl.ds(0, n)], sb)

    def wait(off, n, abuf, bbuf, sa, sb):
        pltpu.make_async_copy(sh1.at[idx1_v.at[pl.ds(off, n)]],
                              abuf.at[pl.ds(0, n)], sa).wait()
        pltpu.make_async_copy(emb2_hbm.at[idx2_v.at[pl.ds(off, n)]],
                              bbuf.at[pl.ds(0, n)], sb).wait()

    def compute(out_off, ngroups, abuf, bbuf):
        for g in range(ngroups):
            row = lane + (g * 16)

            def dstep(k, accs):
                acc0, acc1 = accs
                tbase = k * 4
                p = None
                for dt in range(4):
                    w = jnp.full((16,), tbase + dt, jnp.int32)
                    aw = plsc.load_gather(abuf, [row, w])
                    bw = plsc.load_gather(bbuf, [row, w])
                    prod = (plsc.bitcast(aw, jnp.bfloat16)
                            * plsc.bitcast(bw, jnp.bfloat16))
                    p = prod if p is None else p + prod
                p0, p1 = plsc.unpack(p, format=plsc.PackFormat.INTERLEAVED)
                return (acc0 + p0, acc1 + p1)

            z = jnp.zeros((16,), jnp.float32)
            loop = plsc.parallel_loop(0, _W // 4, 1, unroll=8,
                                      carry=(z, z))(dstep)
            acc0, acc1 = loop
            out_v[pl.ds(out_off + g * 16, 16)] = acc0 + acc1

    start(0, _C, a0, b0, sa0, sb0)

    @pl.loop(0, (_NF - 2) // 2)
    def _pair(i):
        c0 = 2 * i
        start((c0 + 1) * _C, _C, a1, b1, sa1, sb1)
        wait(c0 * _C, _C, a0, b0, sa0, sb0)
        compute(c0 * _C, _G, a0, b0)
        start((c0 + 2) * _C, _C, a0, b0, sa0, sb0)
        wait((c0 + 1) * _C, _C, a1, b1, sa1, sb1)
        compute((c0 + 1) * _C, _G, a1, b1)

    start((_NF - 1) * _C, _C, a1, b1, sa1, sb1)
    wait((_NF - 2) * _C, _C, a0, b0, sa0, sb0)
    compute((_NF - 2) * _C, _G, a0, b0)
    start(_NF * _C, _CR, a0, b0, sa0, sb0)
    wait((_NF - 1) * _C, _C, a1, b1, sa1, sb1)
    compute((_NF - 1) * _C, _G, a1, b1)
    wait(_NF * _C, _CR, a0, b0, sa0, sb0)
    compute(_NF * _C, _CR // 16, a0, b0)

    pltpu.sync_copy(out_v, out_hbm.at[pl.ds(base, _EW)])


def _pack(emb):
    v = lax.bitcast_convert_type(emb, jnp.uint32)
    r = v + jnp.uint32(0x7FFF) + ((v >> 16) & jnp.uint32(1))
    lo = r[:, :_W] >> 16
    hi = r[:, _W:] & jnp.uint32(0xFFFF0000)
    return lax.bitcast_convert_type(lo | hi, jnp.int32)


@jax.jit
def _sc_dot(eidx, emb1, emb2):
    mesh = plsc.VectorSubcoreMesh(core_axis_name="c", subcore_axis_name="s")
    return pl.kernel(
        _body,
        out_type=jax.ShapeDtypeStruct((_N_EDGES,), jnp.float32),
        mesh=mesh,
        compiler_params=pltpu.CompilerParams(needs_layout_passes=False, use_tc_tiling_on_sc=False),
        scratch_types=[
            pltpu.VMEM((_EW,), jnp.int32),
            pltpu.VMEM((_EW,), jnp.int32),
            pltpu.VMEM((_C, _W), jnp.int32),
            pltpu.VMEM((_C, _W), jnp.int32),
            pltpu.VMEM((_C, _W), jnp.int32),
            pltpu.VMEM((_C, _W), jnp.int32),
            pltpu.VMEM((_EW,), jnp.float32),
            pltpu.VMEM_SHARED((_N_NODES, _W), jnp.int32),
            pltpu.SemaphoreType.DMA,
            pltpu.SemaphoreType.DMA,
            pltpu.SemaphoreType.DMA,
            pltpu.SemaphoreType.DMA,
            pltpu.SemaphoreType.DMA,
            pltpu.SemaphoreType.DMA,
        ],
    )(eidx, emb1, emb2)


def kernel(embedding_1, embedding_2, edge_label_index):
    return _sc_dot(edge_label_index.astype(jnp.int32),
                   _pack(embedding_1), _pack(embedding_2))


# speedup vs baseline: 5.8961x; 5.8961x over previous
import functools

import jax
import jax.numpy as jnp
from jax import lax
from jax.experimental import pallas as pl
from jax.experimental.pallas import tpu as pltpu
from jax.experimental.pallas import tpu_sc as plsc

_N_EDGES = 320000
_N_NODES = 10000
_D = 128
_W = _D // 2
_NW = 32
_EW = _N_EDGES // _NW
_C = 128
_NF = _EW // _C
_CR = _EW - _NF * _C
_G = _C // 16


def _body(eidx_hbm, emb1_hbm, emb2_hbm, out_hbm,
          idx1_v, idx2_v, a0, b0, a1, b1, out_v, sh1,
          sa0, sb0, sa1, sb1, st1, st2):
    nc = 2
    wid = lax.axis_index("s") * nc + lax.axis_index("c")
    base = wid * _EW

    sid = lax.axis_index("s")
    srows = _N_NODES // 16
    soff = sid * srows
    cs1 = pltpu.async_copy(emb1_hbm.at[pl.ds(soff, srows)],
                           sh1.at[pl.ds(soff, srows)], st1)

    pltpu.sync_copy(eidx_hbm.at[pl.ds(base, _EW)], idx1_v)
    pltpu.sync_copy(eidx_hbm.at[pl.ds(_N_EDGES + base, _EW)], idx2_v)

    cs1.wait()
    plsc.subcore_barrier()

    lane = lax.iota(jnp.int32, 16)

    def start(off, n, abuf, bbuf, sa, sb):
        pltpu.async_copy(sh1.at[idx1_v.at[pl.ds(off, n)]],
                         abuf.at[pl.ds(0, n)], sa)
        pltpu.async_copy(emb2_hbm.at[idx2_v.at[pl.ds(off, n)]],
                         bbuf.at[pl.ds(0, n)], sb)

    def wait(off, n, abuf, bbuf, sa, sb):
        pltpu.make_async_copy(sh1.at[idx1_v.at[pl.ds(off, n)]],
                              abuf.at[pl.ds(0, n)], sa).wait()
        pltpu.make_async_copy(emb2_hbm.at[idx2_v.at[pl.ds(off, n)]],
                              bbuf.at[pl.ds(0, n)], sb).wait()

    def compute(out_off, ngroups, abuf, bbuf):
        for g in range(ngroups):
            row = lane + (g * 16)

            def dstep(k, accs):
                acc0, acc1 = accs
                tbase = lane + k * 4
                p = None
                for dt in range(4):
                    w = (tbase + dt) & (_W - 1)
                    aw = plsc.load_gather(abuf, [row, w])
                    bw = plsc.load_gather(bbuf, [row, w])
                    prod = (plsc.bitcast(aw, jnp.bfloat16)
                            * plsc.bitcast(bw, jnp.bfloat16))
                    p = prod if p is None else p + prod
                p0, p1 = plsc.unpack(p, format=plsc.PackFormat.INTERLEAVED)
                return (acc0 + p0, acc1 + p1)

            z = jnp.zeros((16,), jnp.float32)
            loop = plsc.parallel_loop(0, _W // 4, 1, unroll=8,
                                      carry=(z, z))(dstep)
            acc0, acc1 = loop
            out_v[pl.ds(out_off + g * 16, 16)] = acc0 + acc1

    start(0, _C, a0, b0, sa0, sb0)

    @pl.loop(0, (_NF - 2) // 2)
    def _pair(i):
        c0 = 2 * i
        start((c0 + 1) * _C, _C, a1, b1, sa1, sb1)
        wait(c0 * _C, _C, a0, b0, sa0, sb0)
        compute(c0 * _C, _G, a0, b0)
        start((c0 + 2) * _C, _C, a0, b0, sa0, sb0)
        wait((c0 + 1) * _C, _C, a1, b1, sa1, sb1)
        compute((c0 + 1) * _C, _G, a1, b1)

    start((_NF - 1) * _C, _C, a1, b1, sa1, sb1)
    wait((_NF - 2) * _C, _C, a0, b0, sa0, sb0)
    compute((_NF - 2) * _C, _G, a0, b0)
    start(_NF * _C, _CR, a0, b0, sa0, sb0)
    wait((_NF - 1) * _C, _C, a1, b1, sa1, sb1)
    compute((_NF - 1) * _C, _G, a1, b1)
    wait(_NF * _C, _CR, a0, b0, sa0, sb0)
    compute(_NF * _C, _CR // 16, a0, b0)

    pltpu.sync_copy(out_v, out_hbm.at[pl.ds(base, _EW)])


def _pack(emb):
    v = lax.bitcast_convert_type(emb, jnp.uint32)
    r = v + jnp.uint32(0x7FFF) + ((v >> 16) & jnp.uint32(1))
    lo = r[:, :_W] >> 16
    hi = r[:, _W:] & jnp.uint32(0xFFFF0000)
    return lax.bitcast_convert_type(lo | hi, jnp.int32)


@jax.jit
def _sc_dot(eidx, emb1, emb2):
    mesh = plsc.VectorSubcoreMesh(core_axis_name="c", subcore_axis_name="s")
    return pl.kernel(
        _body,
        out_type=jax.ShapeDtypeStruct((_N_EDGES,), jnp.float32),
        mesh=mesh,
        compiler_params=pltpu.CompilerParams(needs_layout_passes=False, use_tc_tiling_on_sc=False),
        scratch_types=[
            pltpu.VMEM((_EW,), jnp.int32),
            pltpu.VMEM((_EW,), jnp.int32),
            pltpu.VMEM((_C, _W), jnp.int32),
            pltpu.VMEM((_C, _W), jnp.int32),
            pltpu.VMEM((_C, _W), jnp.int32),
            pltpu.VMEM((_C, _W), jnp.int32),
            pltpu.VMEM((_EW,), jnp.float32),
            pltpu.VMEM_SHARED((_N_NODES, _W), jnp.int32),
            pltpu.SemaphoreType.DMA,
            pltpu.SemaphoreType.DMA,
            pltpu.SemaphoreType.DMA,
            pltpu.SemaphoreType.DMA,
            pltpu.SemaphoreType.DMA,
            pltpu.SemaphoreType.DMA,
        ],
    )(eidx, emb1, emb2)


def kernel(embedding_1, embedding_2, edge_label_index):
    return _sc_dot(edge_label_index.astype(jnp.int32).ravel(),
                   _pack(embedding_1), _pack(embedding_2))
